# baseline (speedup 1.0000x reference)
import functools

import jax
import jax.numpy as jnp
from jax import lax
from jax.experimental import pallas as pl
from jax.experimental.pallas import tpu as pltpu
from jax.experimental.pallas import tpu_sc as plsc

N_NODES = 10000
N_EDGES = 320000
D = 128

NC = 2
NS = 16
NW = NC * NS
CHUNK = 64
CHUNKS_PER_W = 160
HALF = CHUNKS_PER_W // 4
EDGES_PER_W = CHUNK * CHUNKS_PER_W
E_PAD = NW * EDGES_PER_W
ROWS_PER_TILE = 640
ACC_ROWS = NS * ROWS_PER_TILE


E_ROWS = N_EDGES // CHUNK
TAIL_REAL = E_ROWS - (NW - 1) * CHUNKS_PER_W
PAD_ROWS = NW * CHUNKS_PER_W - E_ROWS
DEPTH = 4


def _seg_body(h_hbm, edges_hbm, padc_hbm, out_hbm,
              src_v, dst_v, rows0, rows1, rows2, rows3, acc,
              sem0, sem1, sem2, sem3):
    cid = lax.axis_index("c")
    sid = lax.axis_index("s")
    wid = cid * NS + sid
    rows = [rows0, rows1, rows2, rows3]
    sems = [sem0, sem1, sem2, sem3]

    zvec = jnp.zeros((16,), jnp.float32)

    def zrow(r, _):
        for c in range(8):
            rows0[r, pl.ds(c * 16, 16)] = zvec
        return 0

    lax.fori_loop(0, CHUNK, zrow, 0)
    for i in range(ROWS_PER_TILE // CHUNK):
        pltpu.sync_copy(rows0, acc.at[pl.ds(sid * ROWS_PER_TILE + i * CHUNK, CHUNK)])

    for half in range(4):
        row0 = wid * CHUNKS_PER_W + half * HALF

        @pl.when(wid < NW - 1)
        def _():
            pltpu.sync_copy(edges_hbm.at[1, pl.ds(row0, HALF)], src_v)
            pltpu.sync_copy(edges_hbm.at[0, pl.ds(row0, HALF)], dst_v)

        @pl.when(wid == NW - 1)
        def _():
            pltpu.sync_copy(padc_hbm.at[1, pl.ds(half * HALF, HALF)], src_v)
            pltpu.sync_copy(padc_hbm.at[0, pl.ds(half * HALF, HALF)], dst_v)

        for k in range(DEPTH):
            pltpu.async_copy(h_hbm.at[src_v.at[k]], rows[k], sems[k])

        if half == 0:
            plsc.subcore_barrier()

        def body(g, _):
            c0 = DEPTH * g
            for k in range(DEPTH):
                pltpu.make_async_copy(h_hbm.at[src_v.at[c0 + k]],
                                      rows[k], sems[k]).wait()
                pltpu.sync_copy(rows[k], acc.at[dst_v.at[c0 + k]], add=True)

                @pl.when(c0 + DEPTH + k < HALF)
                def _():
                    pltpu.async_copy(h_hbm.at[src_v.at[c0 + DEPTH + k]],
                                     rows[k], sems[k])

            return 0

        lax.fori_loop(0, HALF // DEPTH, body, 0)

    plsc.subcore_barrier()
    pltpu.sync_copy(acc.at[pl.ds(sid * ROWS_PER_TILE, ROWS_PER_TILE)],
                    out_hbm.at[cid, pl.ds(sid * ROWS_PER_TILE, ROWS_PER_TILE)])


_seg_sum = pl.kernel(
    _seg_body,
    out_type=jax.ShapeDtypeStruct((NC, ACC_ROWS, D), jnp.float32),
    mesh=plsc.VectorSubcoreMesh(core_axis_name="c", subcore_axis_name="s",
                                num_cores=NC, num_subcores=NS),
    scratch_types=[
        pltpu.VMEM((HALF, CHUNK), jnp.int32),
        pltpu.VMEM((HALF, CHUNK), jnp.int32),
        pltpu.VMEM((CHUNK, D), jnp.float32),
        pltpu.VMEM((CHUNK, D), jnp.float32),
        pltpu.VMEM((CHUNK, D), jnp.float32),
        pltpu.VMEM((CHUNK, D), jnp.float32),
        pltpu.VMEM_SHARED((ACC_ROWS, D), jnp.float32),
        pltpu.SemaphoreType.DMA,
        pltpu.SemaphoreType.DMA,
        pltpu.SemaphoreType.DMA,
        pltpu.SemaphoreType.DMA,
    ],
)


ROW_BLK = 2000
GRID = N_NODES // ROW_BLK


def _mid_body(p0_ref, p1_ref, w1_ref, o_ref):
    o_ref[...] = jax.nn.relu(
        lax.dot_general(p0_ref[0] + p1_ref[0], w1_ref[...],
                        (((1,), (1,)), ((), ())),
                        preferred_element_type=jnp.float32))


def _mid_matmul(p, w1):
    return pl.pallas_call(
        _mid_body,
        grid=(GRID,),
        in_specs=[
            pl.BlockSpec((1, ROW_BLK, D), lambda i: (0, i, 0)),
            pl.BlockSpec((1, ROW_BLK, D), lambda i: (1, i, 0)),
            pl.BlockSpec((D, D), lambda i: (0, 0)),
        ],
        out_specs=pl.BlockSpec((ROW_BLK, D), lambda i: (i, 0)),
        out_shape=jax.ShapeDtypeStruct((N_NODES, D), jnp.float32),
    )(p, p, w1)


def _final_body(q0_ref, q1_ref, w2_ref, o_ref):
    o_ref[...] = jax.nn.relu(
        lax.dot_general(q0_ref[0] + q1_ref[0], w2_ref[...],
                        (((1,), (1,)), ((), ())),
                        preferred_element_type=jnp.float32))


def _final_matmul(q, w2):
    return pl.pallas_call(
        _final_body,
        grid=(GRID,),
        in_specs=[
            pl.BlockSpec((1, ROW_BLK, D), lambda i: (0, i, 0)),
            pl.BlockSpec((1, ROW_BLK, D), lambda i: (1, i, 0)),
            pl.BlockSpec((D, D), lambda i: (0, 0)),
        ],
        out_specs=pl.BlockSpec((ROW_BLK, D), lambda i: (i, 0)),
        out_shape=jax.ShapeDtypeStruct((N_NODES, D), jnp.float32),
    )(q, q, w2)


def kernel(X_mask, edge_index, W1, W2):
    edges3 = edge_index.astype(jnp.int32).reshape(2, E_ROWS, CHUNK)
    k = jnp.arange(PAD_ROWS * CHUNK, dtype=jnp.int32)
    pad_const = jnp.stack([N_NODES + k % (ACC_ROWS - N_NODES), k % N_NODES]
                          ).reshape(2, PAD_ROWS, CHUNK)
    padc = jnp.concatenate([edges3[:, E_ROWS - TAIL_REAL:], pad_const], axis=1)

    p = _seg_sum(X_mask, edges3, padc)
    h1 = _mid_matmul(p, W1)
    q = _seg_sum(h1, edges3, padc)
    return _final_matmul(q, W2)
